# baseline (speedup 1.0000x reference)
import functools

import numpy as np
import jax
import jax.numpy as jnp
from jax import lax
from jax.experimental import pallas as pl
from jax.experimental.pallas import tpu as pltpu
from jax.experimental.pallas import tpu_sc as plsc

_B = 16
_H = 224
_W = 224
_PATCH = 14
_PH = _H // _PATCH
_PW = _W // _PATCH
_P = _PH * _PW
_D = 1024
_L = 2
_NSETS = 4
_BP = _B * _P
_KSEL = 5
_BIG = 3.0e38

_EMB_ROWS = 512
_Q_ROWS = 1024


def _pool_matrix() -> np.ndarray:
    m = np.zeros((_P, _P), np.float32)
    for y in range(_PH):
        for x in range(_PW):
            ys = range(max(0, y - 1), min(_PH, y + 2))
            xs = range(max(0, x - 1), min(_PW, x + 2))
            w = 1.0 / (len(ys) * len(xs))
            for yy in ys:
                for xx in xs:
                    m[y * _PW + x, yy * _PW + xx] = w
    return m


def _resize_matrix(in_size: int, out_size: int) -> np.ndarray:
    o = np.arange(out_size, dtype=np.float64)
    sample = (o + 0.5) * (in_size / out_size) - 0.5
    i = np.arange(in_size, dtype=np.float64)
    w = np.maximum(0.0, 1.0 - np.abs(sample[None, :] - i[:, None]))
    w = w / w.sum(axis=0, keepdims=True)
    return np.ascontiguousarray(w.T).astype(np.float32)


_POOL2 = np.kron(np.eye(2, dtype=np.float32), _pool_matrix())
_RH = _resize_matrix(_PH, _H)
_RWT = _resize_matrix(_PW, _W).T.copy()


def _patchify_body(pix_ref, out_ref, sem):
    copies = [
        pltpu.make_async_copy(pix_ref.at[:, c, :, dy],
                              out_ref.at[:, :, :, c, dy], sem)
        for c in range(3) for dy in range(_PATCH)
    ]
    for cp in copies:
        cp.start()
    for cp in copies:
        cp.wait()


def _patchify_call(pix6):
    return pl.pallas_call(
        _patchify_body,
        in_specs=[pl.BlockSpec(memory_space=pl.ANY)],
        out_specs=pl.BlockSpec(memory_space=pl.ANY),
        out_shape=jax.ShapeDtypeStruct(
            (_B, _PH, _PW, 3, _PATCH, _PATCH), jnp.float32),
        scratch_shapes=[pltpu.SemaphoreType.DMA],
    )(pix6)


def _embed_body(p_ref, wp_ref, bp_ref, wl_ref, bl_ref, pool_ref, f_ref, n_ref):
    t = jnp.dot(p_ref[...].astype(jnp.bfloat16), wp_ref[...],
                preferred_element_type=jnp.float32)
    t = (t + bp_ref[...]).astype(jnp.bfloat16)
    pool = pool_ref[...]
    for l in range(_L):
        f = jnp.dot(t, wl_ref[l], preferred_element_type=jnp.float32)
        f = jax.nn.gelu(f + bl_ref[l]).astype(jnp.bfloat16)
        pooled = jnp.dot(pool, f, preferred_element_type=jnp.float32)
        for idx, vb in ((2 * l, f), (2 * l + 1, pooled.astype(jnp.bfloat16))):
            f_ref[idx] = vb
            vr = vb.astype(jnp.float32)
            n_ref[idx] = jnp.sum(vr * vr, axis=1, keepdims=True)


def _embed_call(patches, wp, bp, wl, bl, pool):
    n_blk = _BP // _EMB_ROWS
    return pl.pallas_call(
        _embed_body,
        grid=(n_blk,),
        in_specs=[
            pl.BlockSpec((_EMB_ROWS, patches.shape[1]), lambda i: (i, 0)),
            pl.BlockSpec(wp.shape, lambda i: (0, 0)),
            pl.BlockSpec((1, _D), lambda i: (0, 0)),
            pl.BlockSpec((_L, _D, _D), lambda i: (0, 0, 0)),
            pl.BlockSpec((_L, 1, _D), lambda i: (0, 0, 0)),
            pl.BlockSpec((_EMB_ROWS, _EMB_ROWS), lambda i: (0, 0)),
        ],
        out_specs=[
            pl.BlockSpec((_NSETS, _EMB_ROWS, _D), lambda i: (0, i, 0)),
            pl.BlockSpec((_NSETS, _EMB_ROWS, 1), lambda i: (0, i, 0)),
        ],
        out_shape=[
            jax.ShapeDtypeStruct((_NSETS, _BP, _D), jnp.bfloat16),
            jax.ShapeDtypeStruct((_NSETS, _BP, 1), jnp.float32),
        ],
    )(patches, wp, bp, wl, bl, pool)


def _dist_body(q_ref, k_ref, qn_ref, kn_ref, m_ref):
    qi = pl.program_id(0)
    s = pl.program_id(1)
    jb = pl.program_id(2)
    q = q_ref[0]
    k = k_ref[0]
    g = lax.dot_general(q, k, (((1,), (1,)), ((), ())),
                        preferred_element_type=jnp.float32)
    t = kn_ref[0] - 2.0 * g
    qn = qn_ref[0]
    imgs_per_blk = _Q_ROWS // _P
    row_img = (lax.broadcasted_iota(jnp.int32, (_Q_ROWS, 1), 0) // _P
               + qi * imgs_per_blk)
    for u in range(imgs_per_blk):
        seg = t[:, u * _P:(u + 1) * _P]
        dmin = jnp.min(seg, axis=1, keepdims=True) + qn
        val = jnp.sqrt(jnp.maximum(dmin, 1e-12))
        img = jb * imgs_per_blk + u
        m_ref[u] = jnp.where(row_img == img, _BIG, val)


def _dist_call(f, ncol, nrow):
    n_qblk = _BP // _Q_ROWS
    imgs_per_blk = _Q_ROWS // _P
    return pl.pallas_call(
        _dist_body,
        grid=(n_qblk, _NSETS, n_qblk),
        in_specs=[
            pl.BlockSpec((1, _Q_ROWS, _D), lambda i, s, j: (s, i, 0)),
            pl.BlockSpec((1, _Q_ROWS, _D), lambda i, s, j: (s, j, 0)),
            pl.BlockSpec((1, _Q_ROWS, 1), lambda i, s, j: (s, i, 0)),
            pl.BlockSpec((1, 1, _Q_ROWS), lambda i, s, j: (s * (_BP // _Q_ROWS) + j, 0, 0)),
        ],
        out_specs=pl.BlockSpec((imgs_per_blk, _Q_ROWS, 1),
                               lambda i, s, j: (s * (_BP // _Q_ROWS) + j, i, 0)),
        out_shape=jax.ShapeDtypeStruct((_NSETS * _B, _BP, 1), jnp.float32),
    )(f, f, ncol, nrow)


_SC_WORKERS = 32
_Q_PER_W = _BP // _SC_WORKERS
_Q_GROUP = 16


def _topmin_sc_body(m_hbm, out_hbm, m_v, s_v):
    wid = lax.axis_index("s") * 2 + lax.axis_index("c")
    base = wid * _Q_PER_W
    pltpu.sync_copy(m_hbm.at[:, pl.ds(base, _Q_PER_W)], m_v)
    lane = lax.iota(jnp.int32, 16)
    sel = lane < _KSEL

    def grp_body(g, carry):
        acc = jnp.zeros((16,), jnp.float32)
        for i in range(_Q_GROUP):
            q = g * _Q_GROUP + i
            col = jnp.full((16,), q, jnp.int32)
            tot = jnp.float32(0.0)
            for s in range(_NSETS):
                v = plsc.load_gather(m_v, [s * _B + lane, col])
                sv = jnp.sort(v)
                tot = tot + jnp.sum(jnp.where(sel, sv, 0.0))
            acc = jnp.where(lane == i, tot * (1.0 / (_NSETS * _KSEL)), acc)
        s_v[pl.ds(g * _Q_GROUP, _Q_GROUP)] = acc
        return carry

    lax.fori_loop(0, _Q_PER_W // _Q_GROUP, grp_body, 0)
    pltpu.sync_copy(s_v, out_hbm.at[pl.ds(base, _Q_PER_W)])


@functools.cache
def _topmin_sc_build():
    return pl.kernel(
        _topmin_sc_body,
        out_type=jax.ShapeDtypeStruct((_BP,), jnp.float32),
        mesh=plsc.VectorSubcoreMesh(core_axis_name="c", subcore_axis_name="s"),
        compiler_params=pltpu.CompilerParams(needs_layout_passes=False),
        scratch_types=[
            pltpu.VMEM((_NSETS * _B, _Q_PER_W), jnp.float32),
            pltpu.VMEM((_Q_PER_W,), jnp.float32),
        ],
    )


def _topmin_call(mt):
    return _topmin_sc_build()(mt)


def _resize_body(s_ref, rh_ref, rwt_ref, px_ref, fin_ref):
    sg = s_ref[0]
    t = jnp.dot(rh_ref[...], sg, preferred_element_type=jnp.float32)
    px_ref[0] = jnp.dot(t, rwt_ref[...], preferred_element_type=jnp.float32)
    fin_ref[...] = jnp.max(sg).reshape(1, 1, 1)


def _resize_call(sg, rh, rwt):
    return pl.pallas_call(
        _resize_body,
        grid=(_B,),
        in_specs=[
            pl.BlockSpec((1, _PH, _PW), lambda b: (b, 0, 0)),
            pl.BlockSpec((_H, _PH), lambda b: (0, 0)),
            pl.BlockSpec((_PW, _W), lambda b: (0, 0)),
        ],
        out_specs=[
            pl.BlockSpec((1, _H, _W), lambda b: (b, 0, 0)),
            pl.BlockSpec((1, 1, 1), lambda b: (b, 0, 0)),
        ],
        out_shape=[
            jax.ShapeDtypeStruct((_B, _H, _W), jnp.float32),
            jax.ShapeDtypeStruct((_B, 1, 1), jnp.float32),
        ],
    )(sg, rh, rwt)


def kernel(pixel_values, W_patch, b_patch, W_layers, b_layers):
    pd = 3 * _PATCH * _PATCH
    pix6 = pixel_values.reshape(_B, 3, _PH, _PATCH, _PW, _PATCH)
    patches = _patchify_call(pix6).reshape(_BP, pd)

    f, ncol = _embed_call(
        patches, W_patch.astype(jnp.bfloat16),
        b_patch.reshape(1, _D),
        W_layers.astype(jnp.bfloat16),
        b_layers.reshape(_L, 1, _D),
        jnp.asarray(_POOL2, dtype=jnp.bfloat16),
    )
    nrow = ncol.reshape(_NSETS * (_BP // _Q_ROWS), 1, _Q_ROWS)
    m = _dist_call(f, ncol, nrow)
    mt = m.reshape(_NSETS * _B, _BP)
    scores = _topmin_call(mt)
    sg = scores.reshape(_B, _PH, _PW)
    px, fin = _resize_call(sg, jnp.asarray(_RH), jnp.asarray(_RWT))
    return fin.reshape(_B), px


# speedup vs baseline: 7.5507x; 7.5507x over previous
import functools

import numpy as np
import jax
import jax.numpy as jnp
from jax import lax
from jax.experimental import pallas as pl
from jax.experimental.pallas import tpu as pltpu
from jax.experimental.pallas import tpu_sc as plsc

_B = 16
_H = 224
_W = 224
_PATCH = 14
_PH = _H // _PATCH
_PW = _W // _PATCH
_P = _PH * _PW
_D = 1024
_L = 2
_NSETS = 4
_BP = _B * _P
_KSEL = 5
_BIG = 3.0e38

_EMB_ROWS = 512
_Q_ROWS = 1024


def _pool_matrix() -> np.ndarray:
    m = np.zeros((_P, _P), np.float32)
    for y in range(_PH):
        for x in range(_PW):
            ys = range(max(0, y - 1), min(_PH, y + 2))
            xs = range(max(0, x - 1), min(_PW, x + 2))
            w = 1.0 / (len(ys) * len(xs))
            for yy in ys:
                for xx in xs:
                    m[y * _PW + x, yy * _PW + xx] = w
    return m


def _resize_matrix(in_size: int, out_size: int) -> np.ndarray:
    o = np.arange(out_size, dtype=np.float64)
    sample = (o + 0.5) * (in_size / out_size) - 0.5
    i = np.arange(in_size, dtype=np.float64)
    w = np.maximum(0.0, 1.0 - np.abs(sample[None, :] - i[:, None]))
    w = w / w.sum(axis=0, keepdims=True)
    return np.ascontiguousarray(w.T).astype(np.float32)


_POOL2 = np.kron(np.eye(2, dtype=np.float32), _pool_matrix())
_RH = _resize_matrix(_PH, _H)
_RWT = _resize_matrix(_PW, _W).T.copy()


_PD = 3 * _PATCH * _PATCH
_PAIR_ELEMS = _PW * _PD
_PAIRS_PER_W = (_B * _PH) // 32


def _patchify_sc_body(pix_hbm, out_hbm, idx_v, slab_v, out_v, sem):
    wid = lax.axis_index("s") * 2 + lax.axis_index("c")
    lane = lax.iota(jnp.int32, 16)
    n_vec = _PAIR_ELEMS // 16

    def idx_body(v, carry):
        e = v * 16 + lane
        x = e // _PD
        r = e % _PD
        c = r // (_PATCH * _PATCH)
        rr = r % (_PATCH * _PATCH)
        dy = rr // _PATCH
        dx = rr % _PATCH
        idx_v[pl.ds(v * 16, 16)] = (c * (_PATCH * _H) + dy * _H
                                    + x * _PATCH + dx)
        return carry

    lax.fori_loop(0, n_vec, idx_body, 0)

    def pair_body(pp, carry):
        gp = wid * _PAIRS_PER_W + pp
        b = gp // _PH
        y = gp % _PH
        cps = [
            pltpu.async_copy(
                pix_hbm.at[pl.ds(((b * 3 + c) * _H + _PATCH * y) * _H,
                                 _PATCH * _H)],
                slab_v.at[pl.ds(c * (_PATCH * _H), _PATCH * _H)], sem)
            for c in range(3)
        ]
        for cp in cps:
            cp.wait()

        def mv_body(v, carry2):
            idx = idx_v[pl.ds(v * 16, 16)]
            out_v[pl.ds(v * 16, 16)] = plsc.load_gather(slab_v, [idx])
            return carry2

        lax.fori_loop(0, n_vec, mv_body, 0)
        pltpu.sync_copy(out_v, out_hbm.at[pl.ds(gp * _PAIR_ELEMS,
                                                _PAIR_ELEMS)])
        return carry

    lax.fori_loop(0, _PAIRS_PER_W, pair_body, 0)


@functools.cache
def _patchify_sc_build():
    return pl.kernel(
        _patchify_sc_body,
        out_type=jax.ShapeDtypeStruct((_BP * _PD,), jnp.float32),
        mesh=plsc.VectorSubcoreMesh(core_axis_name="c", subcore_axis_name="s"),
        compiler_params=pltpu.CompilerParams(needs_layout_passes=False),
        scratch_types=[
            pltpu.VMEM((_PAIR_ELEMS,), jnp.int32),
            pltpu.VMEM((_PAIR_ELEMS,), jnp.float32),
            pltpu.VMEM((_PAIR_ELEMS,), jnp.float32),
            pltpu.SemaphoreType.DMA,
        ],
    )


def _patchify_call(pixf):
    return _patchify_sc_build()(pixf)


def _embed_body(p_ref, wp_ref, bp_ref, wl_ref, bl_ref, pool_ref, f_ref, n_ref):
    t = jnp.dot(p_ref[...].astype(jnp.bfloat16),
                wp_ref[...].astype(jnp.bfloat16),
                preferred_element_type=jnp.float32)
    t = (t + bp_ref[...]).astype(jnp.bfloat16)
    pool = pool_ref[...].astype(jnp.bfloat16)
    for l in range(_L):
        f = jnp.dot(t, wl_ref[l].astype(jnp.bfloat16),
                    preferred_element_type=jnp.float32)
        f = jax.nn.gelu(f + bl_ref[l]).astype(jnp.bfloat16)
        pooled = jnp.dot(pool, f, preferred_element_type=jnp.float32)
        for idx, vb in ((2 * l, f), (2 * l + 1, pooled.astype(jnp.bfloat16))):
            f_ref[idx] = vb
            vr = vb.astype(jnp.float32)
            n_ref[idx] = jnp.sum(vr * vr, axis=1, keepdims=True)


def _embed_call(patches, wp, bp, wl, bl, pool):
    n_blk = _BP // _EMB_ROWS
    return pl.pallas_call(
        _embed_body,
        grid=(n_blk,),
        in_specs=[
            pl.BlockSpec((_EMB_ROWS, patches.shape[1]), lambda i: (i, 0)),
            pl.BlockSpec(wp.shape, lambda i: (0, 0)),
            pl.BlockSpec((1, _D), lambda i: (0, 0)),
            pl.BlockSpec((_L, _D, _D), lambda i: (0, 0, 0)),
            pl.BlockSpec((_L, 1, _D), lambda i: (0, 0, 0)),
            pl.BlockSpec((_EMB_ROWS, _EMB_ROWS), lambda i: (0, 0)),
        ],
        out_specs=[
            pl.BlockSpec((_NSETS, _EMB_ROWS, _D), lambda i: (0, i, 0)),
            pl.BlockSpec((_NSETS, _EMB_ROWS, 1), lambda i: (0, i, 0)),
        ],
        out_shape=[
            jax.ShapeDtypeStruct((_NSETS, _BP, _D), jnp.bfloat16),
            jax.ShapeDtypeStruct((_NSETS, _BP, 1), jnp.float32),
        ],
    )(patches, wp, bp, wl, bl, pool)


def _dist_body(q_ref, k_ref, qn_ref, kn_ref, m_ref):
    qi = pl.program_id(0)
    s = pl.program_id(1)
    jb = pl.program_id(2)
    q = q_ref[0]
    k = k_ref[0]
    g = lax.dot_general(q, k, (((1,), (1,)), ((), ())),
                        preferred_element_type=jnp.float32)
    t = kn_ref[0] - 2.0 * g
    qn = qn_ref[0]
    imgs_per_blk = _Q_ROWS // _P
    row_img = (lax.broadcasted_iota(jnp.int32, (_Q_ROWS, 1), 0) // _P
               + qi * imgs_per_blk)
    for u in range(imgs_per_blk):
        seg = t[:, u * _P:(u + 1) * _P]
        dmin = jnp.min(seg, axis=1, keepdims=True) + qn
        val = jnp.sqrt(jnp.maximum(dmin, 1e-12))
        img = jb * imgs_per_blk + u
        m_ref[u] = jnp.where(row_img == img, _BIG, val)


def _dist_call(f, ncol, nrow):
    n_qblk = _BP // _Q_ROWS
    imgs_per_blk = _Q_ROWS // _P
    return pl.pallas_call(
        _dist_body,
        grid=(n_qblk, _NSETS, n_qblk),
        in_specs=[
            pl.BlockSpec((1, _Q_ROWS, _D), lambda i, s, j: (s, i, 0)),
            pl.BlockSpec((1, _Q_ROWS, _D), lambda i, s, j: (s, j, 0)),
            pl.BlockSpec((1, _Q_ROWS, 1), lambda i, s, j: (s, i, 0)),
            pl.BlockSpec((1, 1, _Q_ROWS), lambda i, s, j: (s * (_BP // _Q_ROWS) + j, 0, 0)),
        ],
        out_specs=pl.BlockSpec((imgs_per_blk, _Q_ROWS, 1),
                               lambda i, s, j: (s * (_BP // _Q_ROWS) + j, i, 0)),
        out_shape=jax.ShapeDtypeStruct((_NSETS * _B, _BP, 1), jnp.float32),
    )(f, f, ncol, nrow)


_SC_WORKERS = 32
_Q_PER_W = _BP // _SC_WORKERS
_Q_GROUP = 16


def _topmin_sc_body(m_hbm, out_hbm, m_v, s_v):
    wid = lax.axis_index("s") * 2 + lax.axis_index("c")
    base = wid * _Q_PER_W
    pltpu.sync_copy(m_hbm.at[:, pl.ds(base, _Q_PER_W)], m_v)
    lane = lax.iota(jnp.int32, 16)
    sel = lane < _KSEL

    def grp_body(g, carry):
        acc = jnp.zeros((16,), jnp.float32)
        for i in range(_Q_GROUP):
            q = g * _Q_GROUP + i
            col = jnp.full((16,), q, jnp.int32)
            tot = jnp.float32(0.0)
            for s in range(_NSETS):
                v = plsc.load_gather(m_v, [s * _B + lane, col])
                sv = jnp.sort(v)
                tot = tot + jnp.sum(jnp.where(sel, sv, 0.0))
            acc = jnp.where(lane == i, tot * (1.0 / (_NSETS * _KSEL)), acc)
        s_v[pl.ds(g * _Q_GROUP, _Q_GROUP)] = acc
        return carry

    lax.fori_loop(0, _Q_PER_W // _Q_GROUP, grp_body, 0)
    pltpu.sync_copy(s_v, out_hbm.at[pl.ds(base, _Q_PER_W)])


@functools.cache
def _topmin_sc_build():
    return pl.kernel(
        _topmin_sc_body,
        out_type=jax.ShapeDtypeStruct((_BP,), jnp.float32),
        mesh=plsc.VectorSubcoreMesh(core_axis_name="c", subcore_axis_name="s"),
        compiler_params=pltpu.CompilerParams(needs_layout_passes=False),
        scratch_types=[
            pltpu.VMEM((_NSETS * _B, _Q_PER_W), jnp.float32),
            pltpu.VMEM((_Q_PER_W,), jnp.float32),
        ],
    )


def _topmin_call(mt):
    return _topmin_sc_build()(mt)


def _resize_body(s_ref, rh_ref, rwt_ref, px_ref, fin_ref):
    sg = s_ref[0]
    t = jnp.dot(rh_ref[...], sg, preferred_element_type=jnp.float32)
    px_ref[0] = jnp.dot(t, rwt_ref[...], preferred_element_type=jnp.float32)
    fin_ref[...] = jnp.max(sg).reshape(1, 1, 1)


def _resize_call(sg, rh, rwt):
    return pl.pallas_call(
        _resize_body,
        grid=(_B,),
        in_specs=[
            pl.BlockSpec((1, _PH, _PW), lambda b: (b, 0, 0)),
            pl.BlockSpec((_H, _PH), lambda b: (0, 0)),
            pl.BlockSpec((_PW, _W), lambda b: (0, 0)),
        ],
        out_specs=[
            pl.BlockSpec((1, _H, _W), lambda b: (b, 0, 0)),
            pl.BlockSpec((1, 1, 1), lambda b: (b, 0, 0)),
        ],
        out_shape=[
            jax.ShapeDtypeStruct((_B, _H, _W), jnp.float32),
            jax.ShapeDtypeStruct((_B, 1, 1), jnp.float32),
        ],
    )(sg, rh, rwt)


def kernel(pixel_values, W_patch, b_patch, W_layers, b_layers):
    pd = 3 * _PATCH * _PATCH
    pixf = pixel_values.reshape(_B * 3 * _H * _W)
    patches = _patchify_call(pixf).reshape(_BP, pd)

    f, ncol = _embed_call(
        patches, W_patch,
        b_patch.reshape(1, _D),
        W_layers,
        b_layers.reshape(_L, 1, _D),
        jnp.asarray(_POOL2),
    )
    nrow = ncol.reshape(_NSETS * (_BP // _Q_ROWS), 1, _Q_ROWS)
    m = _dist_call(f, ncol, nrow)
    mt = m.reshape(_NSETS * _B, _BP)
    scores = _topmin_call(mt)
    sg = scores.reshape(_B, _PH, _PW)
    px, fin = _resize_call(sg, jnp.asarray(_RH), jnp.asarray(_RWT))
    return fin.reshape(_B), px
